# baseline (speedup 1.0000x reference)
import functools

import jax
import jax.numpy as jnp
from jax import lax
from jax.experimental import pallas as pl
from jax.experimental.pallas import tpu as pltpu
from jax.experimental.pallas import tpu_sc as plsc

B = 8
N = 8192
G = 512
K = 32
F = 128
GB = 128
XP = 16

NC = 2
NS = 16
NW = NC * NS
ROWS = B * G * K
PER_W = ROWS // NW
CHUNK = 128
NCHUNK = PER_W // CHUNK


def _topk_body(xyz_ref, cen_ref, out_ref):
    Q = N // 4
    cx = cen_ref[0, 0]
    cy = cen_ref[0, 1]
    cz = cen_ref[0, 2]
    g = []
    for k in range(4):
        x = xyz_ref[0, 0:1, k * Q:(k + 1) * Q]
        y = xyz_ref[0, 1:2, k * Q:(k + 1) * Q]
        z = xyz_ref[0, 2:3, k * Q:(k + 1) * Q]
        dx = cx - x
        dy = cy - y
        dz = cz - z
        d2 = (dx * dx + dy * dy) + dz * dz
        d = jnp.sqrt(jnp.maximum(d2, 0.0))
        i = lax.broadcasted_iota(jnp.int32, d.shape, 1) + k * Q
        g.append((d, i))

    def ce(a, b):
        va, ia = a
        vb, ib = b
        swap = (vb < va) | ((vb == va) & (ib < ia))
        lo = (jnp.where(swap, vb, va), jnp.where(swap, ib, ia))
        hi = (jnp.where(swap, va, vb), jnp.where(swap, ia, ib))
        return lo, hi

    g[0], g[1] = ce(g[0], g[1])
    g[2], g[3] = ce(g[2], g[3])
    g[0], g[2] = ce(g[0], g[2])
    g[1], g[3] = ce(g[1], g[3])
    g[1], g[2] = ce(g[1], g[2])
    (v0, i0), (v1, i1), (v2, i2), (v3, i3) = g

    big_i = jnp.int32(N)
    inf = jnp.float32(jnp.inf)
    cols = []
    for _ in range(K):
        m = jnp.min(v0, axis=1, keepdims=True)
        gidx = jnp.min(jnp.where(v0 == m, i0, big_i), axis=1,
                       keepdims=True)
        cols.append(gidx)
        mask = i0 == gidx
        v0 = jnp.where(mask, v1, v0)
        i0 = jnp.where(mask, i1, i0)
        v1 = jnp.where(mask, v2, v1)
        i1 = jnp.where(mask, i2, i1)
        v2 = jnp.where(mask, v3, v2)
        i2 = jnp.where(mask, i3, i2)
        v3 = jnp.where(mask, inf, v3)
    idx_blk = jnp.concatenate(cols, axis=1)
    b = pl.program_id(0)
    out_ref[0] = idx_blk + b * N


def _topk_indices(xyz_t, cen_in):
    return pl.pallas_call(
        _topk_body,
        grid=(B, G // GB),
        in_specs=[
            pl.BlockSpec((1, 3, N), lambda b, g: (b, 0, 0)),
            pl.BlockSpec((1, 3, GB, 1), lambda b, g: (b, 0, g, 0)),
        ],
        out_specs=pl.BlockSpec((1, GB, K), lambda b, g: (b, g, 0)),
        out_shape=jax.ShapeDtypeStruct((B, G, K), jnp.int32),
        compiler_params=pltpu.CompilerParams(
            dimension_semantics=("parallel", "parallel")),
    )(xyz_t, cen_in)


def _sc_gather(feats_flat, xyzp_flat, flat_idx):
    mesh = plsc.VectorSubcoreMesh(core_axis_name="c", subcore_axis_name="s")

    @functools.partial(
        pl.kernel,
        mesh=mesh,
        out_type=[
            jax.ShapeDtypeStruct((ROWS, F), jnp.float32),
            jax.ShapeDtypeStruct((ROWS, XP), jnp.float32),
        ],
        scratch_types=[
            pltpu.VMEM((CHUNK,), jnp.int32),
            pltpu.VMEM((CHUNK, F), jnp.float32),
            pltpu.VMEM((CHUNK, XP), jnp.float32),
            pltpu.SemaphoreType.DMA,
            pltpu.SemaphoreType.DMA,
        ],
        compiler_params=pltpu.CompilerParams(use_tc_tiling_on_sc=False),
    )
    def gather_kernel(feats_hbm, xyzp_hbm, idx_hbm, patch_out, neigh_out,
                      idx_v, frows_v, xrows_v, sem_f, sem_x):
        wid = lax.axis_index("s") * NC + lax.axis_index("c")
        wbase = wid * PER_W

        def chunk_step(t, carry):
            base = pl.multiple_of(wbase + t * CHUNK, CHUNK)
            pltpu.sync_copy(idx_hbm.at[pl.ds(base, CHUNK)], idx_v)
            cp_f = pltpu.async_copy(feats_hbm.at[idx_v], frows_v, sem_f)
            cp_x = pltpu.async_copy(xyzp_hbm.at[idx_v], xrows_v, sem_x)
            cp_f.wait()
            cp_x.wait()
            pltpu.sync_copy(frows_v, patch_out.at[pl.ds(base, CHUNK)])
            pltpu.sync_copy(xrows_v, neigh_out.at[pl.ds(base, CHUNK)])
            return carry

        lax.fori_loop(0, NCHUNK, chunk_step, 0)

    return gather_kernel(feats_flat, xyzp_flat, flat_idx)


def kernel(xyz, feats):
    idx_c = jnp.linspace(0.0, N - 1, G).astype(jnp.int32)
    center = jnp.take(xyz, idx_c, axis=1)

    xyz_t = jnp.transpose(xyz, (0, 2, 1))
    cen_in = jnp.transpose(center, (0, 2, 1))[..., None]

    idx = _topk_indices(xyz_t, cen_in)
    flat_idx = idx.reshape(-1)

    feats_flat = feats.reshape(B * N, F)
    xyzp_flat = jnp.pad(xyz.reshape(B * N, 3), ((0, 0), (0, XP - 3)))
    patch_flat, neigh_flat = _sc_gather(feats_flat, xyzp_flat, flat_idx)

    neighborhood = neigh_flat[:, :3].reshape(B, G, K, 3)
    patch_feats = patch_flat.reshape(B, G, K, F)
    return neighborhood, center, flat_idx, patch_feats


# speedup vs baseline: 1.0112x; 1.0112x over previous
import functools

import jax
import jax.numpy as jnp
from jax import lax
from jax.experimental import pallas as pl
from jax.experimental.pallas import tpu as pltpu
from jax.experimental.pallas import tpu_sc as plsc

B = 8
N = 8192
G = 512
K = 32
F = 128
GB = 128
XP = 16

NC = 2
NS = 16
NW = NC * NS
ROWS = B * G * K
PER_W = ROWS // NW
CHUNK = 128
NCHUNK = PER_W // CHUNK


def _topk_body(xyz_ref, cen_ref, out_ref):
    Q = N // 4
    cx = cen_ref[0, 0]
    cy = cen_ref[0, 1]
    cz = cen_ref[0, 2]
    g = []
    for k in range(4):
        x = xyz_ref[0, 0:1, k * Q:(k + 1) * Q]
        y = xyz_ref[0, 1:2, k * Q:(k + 1) * Q]
        z = xyz_ref[0, 2:3, k * Q:(k + 1) * Q]
        dx = cx - x
        dy = cy - y
        dz = cz - z
        d2 = (dx * dx + dy * dy) + dz * dz
        d = jnp.sqrt(jnp.maximum(d2, 0.0))
        i = lax.broadcasted_iota(jnp.int32, d.shape, 1) + k * Q
        g.append((d, i))

    def ce(a, b):
        va, ia = a
        vb, ib = b
        swap = (vb < va) | ((vb == va) & (ib < ia))
        lo = (jnp.where(swap, vb, va), jnp.where(swap, ib, ia))
        hi = (jnp.where(swap, va, vb), jnp.where(swap, ia, ib))
        return lo, hi

    g[0], g[1] = ce(g[0], g[1])
    g[2], g[3] = ce(g[2], g[3])
    g[0], g[2] = ce(g[0], g[2])
    g[1], g[3] = ce(g[1], g[3])
    g[1], g[2] = ce(g[1], g[2])
    (v0, i0), (v1, i1), (v2, i2), (v3, i3) = g

    big_i = jnp.int32(N)
    inf = jnp.float32(jnp.inf)
    cols = []
    for _ in range(K):
        m = jnp.min(v0, axis=1, keepdims=True)
        gidx = jnp.min(jnp.where(v0 == m, i0, big_i), axis=1,
                       keepdims=True)
        cols.append(gidx)
        mask = i0 == gidx
        v0 = jnp.where(mask, v1, v0)
        i0 = jnp.where(mask, i1, i0)
        v1 = jnp.where(mask, v2, v1)
        i1 = jnp.where(mask, i2, i1)
        v2 = jnp.where(mask, v3, v2)
        i2 = jnp.where(mask, i3, i2)
        v3 = jnp.where(mask, inf, v3)
    idx_blk = jnp.concatenate(cols, axis=1)
    b = pl.program_id(0)
    out_ref[0] = idx_blk + b * N


def _topk_indices(xyz_t, cen_in):
    return pl.pallas_call(
        _topk_body,
        grid=(B, G // GB),
        in_specs=[
            pl.BlockSpec((1, 3, N), lambda b, g: (b, 0, 0)),
            pl.BlockSpec((1, 3, GB, 1), lambda b, g: (b, 0, g, 0)),
        ],
        out_specs=pl.BlockSpec((1, GB, K), lambda b, g: (b, g, 0)),
        out_shape=jax.ShapeDtypeStruct((B, G, K), jnp.int32),
        compiler_params=pltpu.CompilerParams(
            dimension_semantics=("parallel", "parallel")),
    )(xyz_t, cen_in)


def _sc_gather_wide(feats_flat, flat_idx):
    mesh = plsc.VectorSubcoreMesh(core_axis_name="c", subcore_axis_name="s")

    @functools.partial(
        pl.kernel,
        mesh=mesh,
        out_type=jax.ShapeDtypeStruct((ROWS, F), jnp.float32),
        scratch_types=[
            pltpu.VMEM((CHUNK,), jnp.int32),
            pltpu.VMEM((CHUNK, F), jnp.float32),
            pltpu.SemaphoreType.DMA,
        ],
    )
    def gather_kernel(feats_hbm, idx_hbm, patch_out, idx_v, frows_v, sem_f):
        wid = lax.axis_index("s") * NC + lax.axis_index("c")
        wbase = wid * PER_W

        def chunk_step(t, carry):
            base = pl.multiple_of(wbase + t * CHUNK, CHUNK)
            pltpu.sync_copy(idx_hbm.at[pl.ds(base, CHUNK)], idx_v)
            pltpu.async_copy(feats_hbm.at[idx_v], frows_v, sem_f).wait()
            pltpu.sync_copy(frows_v, patch_out.at[pl.ds(base, CHUNK)])
            return carry

        lax.fori_loop(0, NCHUNK, chunk_step, 0)

    return gather_kernel(feats_flat, flat_idx)


def _sc_gather_xyz(xyzp_flat, flat_idx):
    mesh = plsc.VectorSubcoreMesh(core_axis_name="c", subcore_axis_name="s")

    @functools.partial(
        pl.kernel,
        mesh=mesh,
        out_type=jax.ShapeDtypeStruct((ROWS, XP), jnp.float32),
        scratch_types=[
            pltpu.VMEM((CHUNK,), jnp.int32),
            pltpu.VMEM((CHUNK, XP), jnp.float32),
            pltpu.SemaphoreType.DMA,
        ],
        compiler_params=pltpu.CompilerParams(use_tc_tiling_on_sc=False),
    )
    def gather_kernel(xyzp_hbm, idx_hbm, neigh_out, idx_v, xrows_v, sem_x):
        wid = lax.axis_index("s") * NC + lax.axis_index("c")
        wbase = wid * PER_W

        def chunk_step(t, carry):
            base = pl.multiple_of(wbase + t * CHUNK, CHUNK)
            pltpu.sync_copy(idx_hbm.at[pl.ds(base, CHUNK)], idx_v)
            pltpu.async_copy(xyzp_hbm.at[idx_v], xrows_v, sem_x).wait()
            pltpu.sync_copy(xrows_v, neigh_out.at[pl.ds(base, CHUNK)])
            return carry

        lax.fori_loop(0, NCHUNK, chunk_step, 0)

    return gather_kernel(xyzp_flat, flat_idx)


def kernel(xyz, feats):
    idx_c = jnp.linspace(0.0, N - 1, G).astype(jnp.int32)
    center = jnp.take(xyz, idx_c, axis=1)

    xyz_t = jnp.transpose(xyz, (0, 2, 1))
    cen_in = jnp.transpose(center, (0, 2, 1))[..., None]

    idx = _topk_indices(xyz_t, cen_in)
    flat_idx = idx.reshape(-1)

    feats_flat = feats.reshape(B * N, F)
    xyzp_flat = jnp.pad(xyz.reshape(B * N, 3), ((0, 0), (0, XP - 3)))
    patch_flat = _sc_gather_wide(feats_flat, flat_idx)
    neigh_flat = _sc_gather_xyz(xyzp_flat, flat_idx)

    neighborhood = neigh_flat[:, :3].reshape(B, G, K, 3)
    patch_feats = patch_flat.reshape(B, G, K, F)
    return neighborhood, center, flat_idx, patch_feats


# speedup vs baseline: 1.0389x; 1.0274x over previous
import functools

import jax
import jax.numpy as jnp
from jax import lax
from jax.experimental import pallas as pl
from jax.experimental.pallas import tpu as pltpu
from jax.experimental.pallas import tpu_sc as plsc

B = 8
N = 8192
G = 512
K = 32
F = 128
GB = 128
XP = 128

NC = 2
NS = 16
NW = NC * NS
ROWS = B * G * K
PER_W = ROWS // NW
CHUNK = 128
NCHUNK = PER_W // CHUNK


def _topk_body(xyz_ref, cen_ref, out_ref):
    Q = N // 4
    cx = cen_ref[0, 0]
    cy = cen_ref[0, 1]
    cz = cen_ref[0, 2]
    g = []
    for k in range(4):
        x = xyz_ref[0, 0:1, k * Q:(k + 1) * Q]
        y = xyz_ref[0, 1:2, k * Q:(k + 1) * Q]
        z = xyz_ref[0, 2:3, k * Q:(k + 1) * Q]
        dx = cx - x
        dy = cy - y
        dz = cz - z
        d2 = (dx * dx + dy * dy) + dz * dz
        d = jnp.sqrt(jnp.maximum(d2, 0.0))
        i = lax.broadcasted_iota(jnp.int32, d.shape, 1) + k * Q
        g.append((d, i))

    def ce(a, b):
        va, ia = a
        vb, ib = b
        swap = (vb < va) | ((vb == va) & (ib < ia))
        lo = (jnp.where(swap, vb, va), jnp.where(swap, ib, ia))
        hi = (jnp.where(swap, va, vb), jnp.where(swap, ia, ib))
        return lo, hi

    g[0], g[1] = ce(g[0], g[1])
    g[2], g[3] = ce(g[2], g[3])
    g[0], g[2] = ce(g[0], g[2])
    g[1], g[3] = ce(g[1], g[3])
    g[1], g[2] = ce(g[1], g[2])
    (v0, i0), (v1, i1), (v2, i2), (v3, i3) = g

    big_i = jnp.int32(N)
    inf = jnp.float32(jnp.inf)
    cols = []
    for _ in range(K):
        m = jnp.min(v0, axis=1, keepdims=True)
        gidx = jnp.min(jnp.where(v0 == m, i0, big_i), axis=1,
                       keepdims=True)
        cols.append(gidx)
        mask = i0 == gidx
        v0 = jnp.where(mask, v1, v0)
        i0 = jnp.where(mask, i1, i0)
        v1 = jnp.where(mask, v2, v1)
        i1 = jnp.where(mask, i2, i1)
        v2 = jnp.where(mask, v3, v2)
        i2 = jnp.where(mask, i3, i2)
        v3 = jnp.where(mask, inf, v3)
    idx_blk = jnp.concatenate(cols, axis=1)
    b = pl.program_id(0)
    out_ref[0] = idx_blk + b * N


def _topk_indices(xyz_t, cen_in):
    return pl.pallas_call(
        _topk_body,
        grid=(B, G // GB),
        in_specs=[
            pl.BlockSpec((1, 3, N), lambda b, g: (b, 0, 0)),
            pl.BlockSpec((1, 3, GB, 1), lambda b, g: (b, 0, g, 0)),
        ],
        out_specs=pl.BlockSpec((1, GB, K), lambda b, g: (b, g, 0)),
        out_shape=jax.ShapeDtypeStruct((B, G, K), jnp.int32),
        compiler_params=pltpu.CompilerParams(
            dimension_semantics=("parallel", "parallel")),
    )(xyz_t, cen_in)


def _sc_gather(feats_flat, xyzp_flat, flat_idx):
    mesh = plsc.VectorSubcoreMesh(core_axis_name="c", subcore_axis_name="s")

    @functools.partial(
        pl.kernel,
        mesh=mesh,
        out_type=[
            jax.ShapeDtypeStruct((ROWS, F), jnp.float32),
            jax.ShapeDtypeStruct((ROWS, XP), jnp.float32),
        ],
        scratch_types=[
            pltpu.VMEM((CHUNK,), jnp.int32),
            pltpu.VMEM((CHUNK,), jnp.int32),
            pltpu.VMEM((CHUNK, F), jnp.float32),
            pltpu.VMEM((CHUNK, F), jnp.float32),
            pltpu.VMEM((CHUNK, XP), jnp.float32),
            pltpu.VMEM((CHUNK, XP), jnp.float32),
            pltpu.SemaphoreType.DMA,
            pltpu.SemaphoreType.DMA,
            pltpu.SemaphoreType.DMA,
            pltpu.SemaphoreType.DMA,
        ],
    )
    def gather_kernel(feats_hbm, xyzp_hbm, idx_hbm, patch_out, neigh_out,
                      idx_a, idx_b, frows_a, frows_b, xrows_a, xrows_b,
                      sem_fa, sem_fb, sem_xa, sem_xb):
        wid = lax.axis_index("s") * NC + lax.axis_index("c")
        wbase = wid * PER_W

        def pair_step(u, carry):
            base_a = pl.multiple_of(wbase + (2 * u) * CHUNK, CHUNK)
            base_b = pl.multiple_of(wbase + (2 * u + 1) * CHUNK, CHUNK)
            pltpu.sync_copy(idx_hbm.at[pl.ds(base_a, CHUNK)], idx_a)
            cp_fa = pltpu.async_copy(feats_hbm.at[idx_a], frows_a, sem_fa)
            cp_xa = pltpu.async_copy(xyzp_hbm.at[idx_a], xrows_a, sem_xa)
            pltpu.sync_copy(idx_hbm.at[pl.ds(base_b, CHUNK)], idx_b)
            cp_fb = pltpu.async_copy(feats_hbm.at[idx_b], frows_b, sem_fb)
            cp_xb = pltpu.async_copy(xyzp_hbm.at[idx_b], xrows_b, sem_xb)
            cp_fa.wait()
            cp_xa.wait()
            pltpu.sync_copy(frows_a, patch_out.at[pl.ds(base_a, CHUNK)])
            pltpu.sync_copy(xrows_a, neigh_out.at[pl.ds(base_a, CHUNK)])
            cp_fb.wait()
            cp_xb.wait()
            pltpu.sync_copy(frows_b, patch_out.at[pl.ds(base_b, CHUNK)])
            pltpu.sync_copy(xrows_b, neigh_out.at[pl.ds(base_b, CHUNK)])
            return carry

        lax.fori_loop(0, NCHUNK // 2, pair_step, 0)

    return gather_kernel(feats_flat, xyzp_flat, flat_idx)


def kernel(xyz, feats):
    idx_c = jnp.linspace(0.0, N - 1, G).astype(jnp.int32)
    center = jnp.take(xyz, idx_c, axis=1)

    xyz_t = jnp.transpose(xyz, (0, 2, 1))
    cen_in = jnp.transpose(center, (0, 2, 1))[..., None]

    idx = _topk_indices(xyz_t, cen_in)
    flat_idx = idx.reshape(-1)

    feats_flat = feats.reshape(B * N, F)
    xyzp_flat = jnp.pad(xyz.reshape(B * N, 3), ((0, 0), (0, XP - 3)))
    patch_flat, neigh_flat = _sc_gather(feats_flat, xyzp_flat, flat_idx)

    neighborhood = neigh_flat[:, :3].reshape(B, G, K, 3)
    patch_feats = patch_flat.reshape(B, G, K, F)
    return neighborhood, center, flat_idx, patch_feats


# speedup vs baseline: 1.0790x; 1.0386x over previous
import functools

import jax
import jax.numpy as jnp
from jax import lax
from jax.experimental import pallas as pl
from jax.experimental.pallas import tpu as pltpu
from jax.experimental.pallas import tpu_sc as plsc

B = 8
N = 8192
G = 512
K = 32
F = 128
GB = 128
XP = 128

NC = 2
NS = 16
NW = NC * NS
ROWS = B * G * K
PER_W = ROWS // NW
CHUNK = 128
NCHUNK = PER_W // CHUNK


def _topk_body(xyz_ref, cen_ref, out_ref):
    S = 8
    Q = N // S
    cx = cen_ref[0, 0]
    cy = cen_ref[0, 1]
    cz = cen_ref[0, 2]
    g = []
    for k in range(S):
        x = xyz_ref[0, 0:1, k * Q:(k + 1) * Q]
        y = xyz_ref[0, 1:2, k * Q:(k + 1) * Q]
        z = xyz_ref[0, 2:3, k * Q:(k + 1) * Q]
        dx = cx - x
        dy = cy - y
        dz = cz - z
        d2 = (dx * dx + dy * dy) + dz * dz
        d = jnp.sqrt(jnp.maximum(d2, 0.0))
        i = lax.broadcasted_iota(jnp.int32, d.shape, 1) + k * Q
        g.append((d, i))

    def ce(a, b):
        va, ia = a
        vb, ib = b
        swap = (vb < va) | ((vb == va) & (ib < ia))
        lo = (jnp.where(swap, vb, va), jnp.where(swap, ib, ia))
        hi = (jnp.where(swap, va, vb), jnp.where(swap, ia, ib))
        return lo, hi

    network = [(0, 1), (2, 3), (4, 5), (6, 7),
               (0, 2), (1, 3), (4, 6), (5, 7),
               (1, 2), (5, 6),
               (0, 4), (1, 5), (2, 6), (3, 7),
               (2, 4), (3, 5),
               (1, 2), (3, 4), (5, 6)]
    for a, b in network:
        g[a], g[b] = ce(g[a], g[b])
    vs = [p[0] for p in g]
    ix = [p[1] for p in g]

    big_i = jnp.int32(N)
    inf = jnp.float32(jnp.inf)
    cols = []
    for _ in range(K):
        m = jnp.min(vs[0], axis=1, keepdims=True)
        gidx = jnp.min(jnp.where(vs[0] == m, ix[0], big_i), axis=1,
                       keepdims=True)
        cols.append(gidx)
        mask = ix[0] == gidx
        for s in range(S - 1):
            vs[s] = jnp.where(mask, vs[s + 1], vs[s])
            ix[s] = jnp.where(mask, ix[s + 1], ix[s])
        vs[S - 1] = jnp.where(mask, inf, vs[S - 1])
    idx_blk = jnp.concatenate(cols, axis=1)
    b = pl.program_id(0)
    out_ref[0] = idx_blk + b * N


def _topk_indices(xyz_t, cen_in):
    return pl.pallas_call(
        _topk_body,
        grid=(B, G // GB),
        in_specs=[
            pl.BlockSpec((1, 3, N), lambda b, g: (b, 0, 0)),
            pl.BlockSpec((1, 3, GB, 1), lambda b, g: (b, 0, g, 0)),
        ],
        out_specs=pl.BlockSpec((1, GB, K), lambda b, g: (b, g, 0)),
        out_shape=jax.ShapeDtypeStruct((B, G, K), jnp.int32),
        compiler_params=pltpu.CompilerParams(
            dimension_semantics=("parallel", "parallel")),
    )(xyz_t, cen_in)


def _sc_gather(feats_flat, xyzp_flat, flat_idx):
    mesh = plsc.VectorSubcoreMesh(core_axis_name="c", subcore_axis_name="s")

    @functools.partial(
        pl.kernel,
        mesh=mesh,
        out_type=[
            jax.ShapeDtypeStruct((ROWS, F), jnp.float32),
            jax.ShapeDtypeStruct((ROWS, XP), jnp.float32),
        ],
        scratch_types=[
            pltpu.VMEM((CHUNK,), jnp.int32),
            pltpu.VMEM((CHUNK,), jnp.int32),
            pltpu.VMEM((CHUNK, F), jnp.float32),
            pltpu.VMEM((CHUNK, F), jnp.float32),
            pltpu.VMEM((CHUNK, XP), jnp.float32),
            pltpu.VMEM((CHUNK, XP), jnp.float32),
            pltpu.SemaphoreType.DMA,
            pltpu.SemaphoreType.DMA,
            pltpu.SemaphoreType.DMA,
            pltpu.SemaphoreType.DMA,
        ],
    )
    def gather_kernel(feats_hbm, xyzp_hbm, idx_hbm, patch_out, neigh_out,
                      idx_a, idx_b, frows_a, frows_b, xrows_a, xrows_b,
                      sem_fa, sem_fb, sem_xa, sem_xb):
        wid = lax.axis_index("s") * NC + lax.axis_index("c")
        wbase = wid * PER_W

        def pair_step(u, carry):
            base_a = pl.multiple_of(wbase + (2 * u) * CHUNK, CHUNK)
            base_b = pl.multiple_of(wbase + (2 * u + 1) * CHUNK, CHUNK)
            pltpu.sync_copy(idx_hbm.at[pl.ds(base_a, CHUNK)], idx_a)
            cp_fa = pltpu.async_copy(feats_hbm.at[idx_a], frows_a, sem_fa)
            cp_xa = pltpu.async_copy(xyzp_hbm.at[idx_a], xrows_a, sem_xa)
            pltpu.sync_copy(idx_hbm.at[pl.ds(base_b, CHUNK)], idx_b)
            cp_fb = pltpu.async_copy(feats_hbm.at[idx_b], frows_b, sem_fb)
            cp_xb = pltpu.async_copy(xyzp_hbm.at[idx_b], xrows_b, sem_xb)
            cp_fa.wait()
            cp_xa.wait()
            pltpu.sync_copy(frows_a, patch_out.at[pl.ds(base_a, CHUNK)])
            pltpu.sync_copy(xrows_a, neigh_out.at[pl.ds(base_a, CHUNK)])
            cp_fb.wait()
            cp_xb.wait()
            pltpu.sync_copy(frows_b, patch_out.at[pl.ds(base_b, CHUNK)])
            pltpu.sync_copy(xrows_b, neigh_out.at[pl.ds(base_b, CHUNK)])
            return carry

        lax.fori_loop(0, NCHUNK // 2, pair_step, 0)

    return gather_kernel(feats_flat, xyzp_flat, flat_idx)


def kernel(xyz, feats):
    idx_c = jnp.linspace(0.0, N - 1, G).astype(jnp.int32)
    center = jnp.take(xyz, idx_c, axis=1)

    xyz_t = jnp.transpose(xyz, (0, 2, 1))
    cen_in = jnp.transpose(center, (0, 2, 1))[..., None]

    idx = _topk_indices(xyz_t, cen_in)
    flat_idx = idx.reshape(-1)

    feats_flat = feats.reshape(B * N, F)
    xyzp_flat = jnp.pad(xyz.reshape(B * N, 3), ((0, 0), (0, XP - 3)))
    patch_flat, neigh_flat = _sc_gather(feats_flat, xyzp_flat, flat_idx)

    neighborhood = neigh_flat[:, :3].reshape(B, G, K, 3)
    patch_feats = patch_flat.reshape(B, G, K, F)
    return neighborhood, center, flat_idx, patch_feats
